# baseline (speedup 1.0000x reference)
import functools

import jax
import jax.numpy as jnp
from jax import lax
from jax.experimental import pallas as pl
from jax.experimental.pallas import tpu as pltpu
from jax.experimental.pallas import tpu_sc as plsc

N = 10000
E = 320000
D = 128
HD = D // 2
NC = 2
NS = 16
CHW = 128
CH_N = 80
CH_T = NC * CH_N
NBUF = 5
DW = 64
EPAD = NC * NS * CH_N * CHW
NPAD = 10112
ROWS_PER_TILE = NPAD // NS

_MESH = dict(core_axis_name="c", subcore_axis_name="s", num_cores=NC,
             num_subcores=NS)


def _sc_deg(dsti, ones, zeros_h):
    @functools.partial(
        pl.kernel,
        out_type=jax.ShapeDtypeStruct((NC, NPAD, DW), jnp.float32),
        mesh=plsc.VectorSubcoreMesh(**_MESH),
        scratch_types=[
            pltpu.VMEM((CH_N, CHW), jnp.int32),
            pltpu.VMEM((CHW, DW), jnp.float32),
            pltpu.VMEM_SHARED((NPAD, DW), jnp.float32),
            pltpu.SemaphoreType.DMA,
        ],
        compiler_params=pltpu.CompilerParams(use_tc_tiling_on_sc=False),
    )
    def k(dsti_hbm, ones_hbm, z_hbm, out_hbm, di_v, ones_v, acc_sh, sem):
        cid = lax.axis_index("c")
        sid = lax.axis_index("s")
        pltpu.sync_copy(dsti_hbm.at[sid].at[pl.ds(cid * CH_N, CH_N)], di_v)
        pltpu.sync_copy(ones_hbm, ones_v)

        base = sid * ROWS_PER_TILE
        pltpu.sync_copy(z_hbm, acc_sh.at[pl.ds(base, ROWS_PER_TILE)])
        plsc.subcore_barrier()

        @pl.loop(0, CH_N, step=16)
        def _(j):
            for i in range(16):
                pltpu.async_copy(ones_v, acc_sh.at[di_v.at[j + i]], sem,
                                 add=True)
            for i in range(16):
                pltpu.make_async_copy(z_hbm.at[pl.ds(0, CHW)], ones_v,
                                      sem).wait()

        plsc.subcore_barrier()
        pltpu.sync_copy(acc_sh.at[pl.ds(base, ROWS_PER_TILE)],
                        out_hbm.at[cid].at[pl.ds(base, ROWS_PER_TILE)])

    return k(dsti, ones, zeros_h)


def _sc_agg(vals2, srci, dsti, zeros_h):
    @functools.partial(
        pl.kernel,
        out_type=jax.ShapeDtypeStruct((NC, NPAD, HD), jnp.float32),
        mesh=plsc.VectorSubcoreMesh(**_MESH),
        scratch_types=(
            [pltpu.VMEM((CH_T, CHW), jnp.int32),
             pltpu.VMEM((CH_T, CHW), jnp.int32)]
            + [pltpu.VMEM((CHW, HD), jnp.float32) for _ in range(NBUF)]
            + [pltpu.VMEM_SHARED((NPAD, HD), jnp.float32)]
            + [pltpu.SemaphoreType.DMA for _ in range(NBUF)]
            + [pltpu.SemaphoreType.DMA for _ in range(NBUF)]
        ),
        compiler_params=pltpu.CompilerParams(use_tc_tiling_on_sc=False),
    )
    def k(vals_hbm, srci_hbm, dsti_hbm, z_hbm, out_hbm, si_v, di_v, *rest):
        gbufs = rest[:NBUF]
        acc_sh = rest[NBUF]
        sems = rest[NBUF + 1:NBUF + 1 + NBUF]
        ssems = rest[NBUF + 1 + NBUF:]
        cid = lax.axis_index("c")
        sid = lax.axis_index("s")

        def fire_g(c, b):
            pltpu.async_copy(vals_hbm.at[cid].at[si_v.at[c]], gbufs[b],
                             sems[b])

        def wait_g(b):
            pltpu.make_async_copy(z_hbm.at[pl.ds(0, CHW)], gbufs[b],
                                  sems[b]).wait()

        def fire_s(c, b):
            pltpu.async_copy(gbufs[b], acc_sh.at[di_v.at[c]], ssems[b],
                             add=True)

        def wait_s(b):
            pltpu.make_async_copy(z_hbm.at[pl.ds(0, CHW)], gbufs[b],
                                  ssems[b]).wait()

        pltpu.sync_copy(srci_hbm.at[sid], si_v)
        for b in range(NBUF):
            fire_g(b, b)

        pltpu.sync_copy(dsti_hbm.at[sid], di_v)
        base = sid * ROWS_PER_TILE
        pltpu.sync_copy(z_hbm, acc_sh.at[pl.ds(base, ROWS_PER_TILE)])
        plsc.subcore_barrier()

        @pl.loop(0, CH_T - NBUF, step=NBUF)
        def _(j):
            for b in range(NBUF):
                wait_g(b)
                fire_s(j + b, b)
            for b in range(NBUF):
                wait_s(b)
                fire_g(j + NBUF + b, b)

        for b in range(NBUF):
            wait_g(b)
            fire_s(CH_T - NBUF + b, b)
        for b in range(NBUF):
            wait_s(b)

        plsc.subcore_barrier()
        pltpu.sync_copy(acc_sh.at[pl.ds(base, ROWS_PER_TILE)],
                        out_hbm.at[cid].at[pl.ds(base, ROWS_PER_TILE)])

    return k(vals2, srci, dsti, zeros_h)


def _dinv(degp_ref):
    deg = 1.0 + degp_ref[0, :, 0:1] + degp_ref[1, :, 0:1]
    return lax.rsqrt(deg)


def _tc_prep_body(x_ref, degp_ref, xs_ref):
    xs = x_ref[...] * _dinv(degp_ref)
    xs_ref[0, :, :] = xs[:, :HD]
    xs_ref[1, :, :] = xs[:, HD:]


def _tc_prep(x_pad, degp):
    return pl.pallas_call(
        _tc_prep_body,
        out_shape=jax.ShapeDtypeStruct((NC, NPAD, HD), jnp.float32),
    )(x_pad, degp)


def _tc_mid_body(a_ref, xs_ref, degp_ref, w, b, hs_ref):
    dinv = _dinv(degp_ref)
    a = jnp.concatenate([a_ref[0], a_ref[1]], axis=-1)
    xs = jnp.concatenate([xs_ref[0], xs_ref[1]], axis=-1)
    p = dinv * (a + xs)
    h = jnp.dot(p, w[...], preferred_element_type=jnp.float32) + b[...]
    hs = dinv * jnp.maximum(h, 0.0)
    hs_ref[0, :, :] = hs[:, :HD]
    hs_ref[1, :, :] = hs[:, HD:]


def _tc_mid(a, xs, degp, w, b):
    return pl.pallas_call(
        _tc_mid_body,
        out_shape=jax.ShapeDtypeStruct((NC, NPAD, HD), jnp.float32),
    )(a, xs, degp, w, b)


def _tc_post_body(a_ref, hs_ref, degp_ref, w, b, out_ref):
    dinv = _dinv(degp_ref)
    a = jnp.concatenate([a_ref[0], a_ref[1]], axis=-1)
    hs = jnp.concatenate([hs_ref[0], hs_ref[1]], axis=-1)
    p = dinv * (a + hs)
    z = jnp.dot(p, w[...], preferred_element_type=jnp.float32) + b[...]
    m = jnp.max(z, axis=-1, keepdims=True)
    zm = z - m
    lse = jnp.log(jnp.sum(jnp.exp(zm), axis=-1, keepdims=True))
    out_ref[...] = zm - lse


def _tc_post(a, hs, degp, w, b):
    return pl.pallas_call(
        _tc_post_body,
        out_shape=jax.ShapeDtypeStruct((NPAD, D), jnp.float32),
    )(a, hs, degp, w, b)


def kernel(x, edge_index, W1, b1, W2, b2):
    src = edge_index[0].astype(jnp.int32)
    dst = edge_index[1].astype(jnp.int32)
    pad = jnp.full((EPAD - E,), N, jnp.int32)
    src_t = jnp.concatenate([src, pad]).reshape(NS, CH_T, CHW)
    dst_t = jnp.concatenate([dst, pad]).reshape(NS, CH_T, CHW)
    x_pad = jnp.pad(x, ((0, NPAD - N), (0, 0)))
    zeros_h = jnp.zeros((ROWS_PER_TILE, HD), jnp.float32)
    ones = jnp.ones((CHW, DW), jnp.float32)
    b1r = b1.reshape(1, D)
    b2r = b2.reshape(1, D)

    degp = _sc_deg(dst_t, ones, zeros_h)
    xs = _tc_prep(x_pad, degp)
    agg = _sc_agg(xs, src_t, dst_t, zeros_h)
    hs = _tc_mid(agg, xs, degp, W1, b1r)
    agg2 = _sc_agg(hs, src_t, dst_t, zeros_h)
    out = _tc_post(agg2, hs, degp, W2, b2r)
    return out[:N]


# speedup vs baseline: 1.0413x; 1.0413x over previous
import functools

import jax
import jax.numpy as jnp
from jax import lax
from jax.experimental import pallas as pl
from jax.experimental.pallas import tpu as pltpu
from jax.experimental.pallas import tpu_sc as plsc

N = 10000
E = 320000
D = 128
HD = D // 2
NC = 2
NS = 16
CHW = 128
CH_N = 80
CH_T = NC * CH_N
NBUF = 5
DW = 64
EPAD = NC * NS * CH_N * CHW
NPAD = 10112
ROWS_PER_TILE = NPAD // NS

_MESH = dict(core_axis_name="c", subcore_axis_name="s", num_cores=NC,
             num_subcores=NS)


def _sc_deg(dsti, ones, zeros_h):
    @functools.partial(
        pl.kernel,
        out_type=jax.ShapeDtypeStruct((NC, NPAD, DW), jnp.float32),
        mesh=plsc.VectorSubcoreMesh(**_MESH),
        scratch_types=[
            pltpu.VMEM((CH_N, CHW), jnp.int32),
            pltpu.VMEM((CHW, DW), jnp.float32),
            pltpu.VMEM_SHARED((NPAD, DW), jnp.float32),
            pltpu.SemaphoreType.DMA,
        ],
        compiler_params=pltpu.CompilerParams(use_tc_tiling_on_sc=False),
    )
    def k(dsti_hbm, ones_hbm, z_hbm, out_hbm, di_v, ones_v, acc_sh, sem):
        cid = lax.axis_index("c")
        sid = lax.axis_index("s")
        pltpu.sync_copy(dsti_hbm.at[sid].at[pl.ds(cid * CH_N, CH_N)], di_v)
        pltpu.sync_copy(ones_hbm, ones_v)

        base = sid * ROWS_PER_TILE
        pltpu.sync_copy(z_hbm, acc_sh.at[pl.ds(base, ROWS_PER_TILE)])
        plsc.subcore_barrier()

        @pl.loop(0, CH_N, step=16)
        def _(j):
            for i in range(16):
                pltpu.async_copy(ones_v, acc_sh.at[di_v.at[j + i]], sem,
                                 add=True)
            for i in range(16):
                pltpu.make_async_copy(z_hbm.at[pl.ds(0, CHW)], ones_v,
                                      sem).wait()

        plsc.subcore_barrier()
        pltpu.sync_copy(acc_sh.at[pl.ds(base, ROWS_PER_TILE)],
                        out_hbm.at[cid].at[pl.ds(base, ROWS_PER_TILE)])

    return k(dsti, ones, zeros_h)


def _sc_agg(vals2, srci, dsti, zeros_h):
    @functools.partial(
        pl.kernel,
        out_type=jax.ShapeDtypeStruct((NC, NPAD, HD), jnp.float32),
        mesh=plsc.VectorSubcoreMesh(**_MESH),
        scratch_types=(
            [pltpu.VMEM((CH_T, CHW), jnp.int32),
             pltpu.VMEM((CH_T, CHW), jnp.int32)]
            + [pltpu.VMEM((CHW, HD), jnp.float32) for _ in range(NBUF)]
            + [pltpu.VMEM_SHARED((NPAD, HD), jnp.float32)]
            + [pltpu.SemaphoreType.DMA for _ in range(NBUF)]
            + [pltpu.SemaphoreType.DMA for _ in range(NBUF)]
        ),
        compiler_params=pltpu.CompilerParams(use_tc_tiling_on_sc=False),
    )
    def k(vals_hbm, srci_hbm, dsti_hbm, z_hbm, out_hbm, si_v, di_v, *rest):
        gbufs = rest[:NBUF]
        acc_sh = rest[NBUF]
        sems = rest[NBUF + 1:NBUF + 1 + NBUF]
        ssems = rest[NBUF + 1 + NBUF:]
        cid = lax.axis_index("c")
        sid = lax.axis_index("s")

        def fire_g(c, b):
            pltpu.async_copy(vals_hbm.at[cid].at[si_v.at[c]], gbufs[b],
                             sems[b])

        def wait_g(b):
            pltpu.make_async_copy(z_hbm.at[pl.ds(0, CHW)], gbufs[b],
                                  sems[b]).wait()

        def fire_s(c, b):
            pltpu.async_copy(gbufs[b], acc_sh.at[di_v.at[c]], ssems[b],
                             add=True)

        def wait_s(b):
            pltpu.make_async_copy(z_hbm.at[pl.ds(0, CHW)], gbufs[b],
                                  ssems[b]).wait()

        pltpu.sync_copy(srci_hbm.at[sid], si_v)
        for b in range(NBUF):
            fire_g(b, b)

        pltpu.sync_copy(dsti_hbm.at[sid], di_v)
        base = sid * ROWS_PER_TILE
        pltpu.sync_copy(z_hbm, acc_sh.at[pl.ds(base, ROWS_PER_TILE)])
        plsc.subcore_barrier()

        @pl.loop(0, CH_T - NBUF, step=NBUF)
        def _(j):
            for b in range(NBUF):
                c = j + b
                wait_g(b)
                fire_s(c, b)
                wait_s(b)
                fire_g(c + NBUF, b)

        for b in range(NBUF):
            wait_g(b)
            fire_s(CH_T - NBUF + b, b)
            wait_s(b)

        plsc.subcore_barrier()
        pltpu.sync_copy(acc_sh.at[pl.ds(base, ROWS_PER_TILE)],
                        out_hbm.at[cid].at[pl.ds(base, ROWS_PER_TILE)])

    return k(vals2, srci, dsti, zeros_h)


def _dinv(degp_ref):
    deg = 1.0 + degp_ref[0, :, 0:1] + degp_ref[1, :, 0:1]
    return lax.rsqrt(deg)


def _tc_prep_body(x_ref, degp_ref, xs_ref):
    xs = x_ref[...] * _dinv(degp_ref)
    xs_ref[0, :, :] = xs[:, :HD]
    xs_ref[1, :, :] = xs[:, HD:]


def _tc_prep(x_pad, degp):
    return pl.pallas_call(
        _tc_prep_body,
        out_shape=jax.ShapeDtypeStruct((NC, NPAD, HD), jnp.float32),
    )(x_pad, degp)


def _tc_mid_body(a_ref, xs_ref, degp_ref, w, b, hs_ref):
    dinv = _dinv(degp_ref)
    a = jnp.concatenate([a_ref[0], a_ref[1]], axis=-1)
    xs = jnp.concatenate([xs_ref[0], xs_ref[1]], axis=-1)
    p = dinv * (a + xs)
    h = jnp.dot(p, w[...], preferred_element_type=jnp.float32) + b[...]
    hs = dinv * jnp.maximum(h, 0.0)
    hs_ref[0, :, :] = hs[:, :HD]
    hs_ref[1, :, :] = hs[:, HD:]


def _tc_mid(a, xs, degp, w, b):
    return pl.pallas_call(
        _tc_mid_body,
        out_shape=jax.ShapeDtypeStruct((NC, NPAD, HD), jnp.float32),
    )(a, xs, degp, w, b)


def _tc_post_body(a_ref, hs_ref, degp_ref, w, b, out_ref):
    dinv = _dinv(degp_ref)
    a = jnp.concatenate([a_ref[0], a_ref[1]], axis=-1)
    hs = jnp.concatenate([hs_ref[0], hs_ref[1]], axis=-1)
    p = dinv * (a + hs)
    z = jnp.dot(p, w[...], preferred_element_type=jnp.float32) + b[...]
    m = jnp.max(z, axis=-1, keepdims=True)
    zm = z - m
    lse = jnp.log(jnp.sum(jnp.exp(zm), axis=-1, keepdims=True))
    out_ref[...] = zm - lse


def _tc_post(a, hs, degp, w, b):
    return pl.pallas_call(
        _tc_post_body,
        out_shape=jax.ShapeDtypeStruct((NPAD, D), jnp.float32),
    )(a, hs, degp, w, b)


def kernel(x, edge_index, W1, b1, W2, b2):
    src = edge_index[0].astype(jnp.int32)
    dst = edge_index[1].astype(jnp.int32)
    pad = jnp.full((EPAD - E,), N, jnp.int32)
    src_t = jnp.concatenate([src, pad]).reshape(NS, CH_T, CHW)
    dst_t = jnp.concatenate([dst, pad]).reshape(NS, CH_T, CHW)
    x_pad = jnp.pad(x, ((0, NPAD - N), (0, 0)))
    zeros_h = jnp.zeros((ROWS_PER_TILE, HD), jnp.float32)
    ones = jnp.ones((CHW, DW), jnp.float32)
    b1r = b1.reshape(1, D)
    b2r = b2.reshape(1, D)

    degp = _sc_deg(dst_t, ones, zeros_h)
    xs = _tc_prep(x_pad, degp)
    agg = _sc_agg(xs, src_t, dst_t, zeros_h)
    hs = _tc_mid(agg, xs, degp, W1, b1r)
    agg2 = _sc_agg(hs, src_t, dst_t, zeros_h)
    out = _tc_post(agg2, hs, degp, W2, b2r)
    return out[:N]
